# Initial kernel scaffold; baseline (speedup 1.0000x reference)
import jax
import jax.numpy as jnp
from jax.experimental import pallas as pl


def kernel(x, edge_index, W1, b1, W2, b2, Wp, bp):
    raise NotImplementedError("write your pallas kernel here")


# speedup vs baseline: 21.7960x; 21.7960x over previous
import functools

import jax
import jax.numpy as jnp
from jax import lax
from jax.experimental import pallas as pl
from jax.experimental.pallas import tpu as pltpu
from jax.experimental.pallas import tpu_sc as plsc

N = 10000
D = 128
NP = 10240
NC, NS, LANES = 2, 16, 16
NW = NC * NS
CB = 128
RPT = NP // NS
BR = 1280
G = NP // BR

_MESH = dict(core_axis_name="c", subcore_axis_name="s")


@functools.lru_cache(maxsize=None)
def _sc_deg(ec):
    @functools.partial(
        pl.kernel,
        out_type=jax.ShapeDtypeStruct((NC, NP, LANES), jnp.float32),
        mesh=plsc.VectorSubcoreMesh(**_MESH),
        scratch_types=[
            pltpu.VMEM((ec, CB), jnp.int32),
            pltpu.VMEM((CB, LANES), jnp.float32),
            pltpu.VMEM_SHARED((NP, LANES), jnp.float32),
        ],
    )
    def deg_kernel(dst_hbm, deg_hbm, dst_v, ones_v, deg_sh):
        cid = lax.axis_index("c")
        sid = lax.axis_index("s")
        w = cid * NS + sid
        base = sid * RPT

        def fill(val):
            def body(i, _):
                ones_v[i] = jnp.full((LANES,), val, jnp.float32)
                return 0
            lax.fori_loop(0, CB, body, 0)

        fill(0.0)
        for k in range(RPT // CB):
            pltpu.sync_copy(ones_v, deg_sh.at[pl.ds(base + k * CB, CB)])
        fill(1.0)
        plsc.subcore_barrier()

        pltpu.sync_copy(dst_hbm.at[w], dst_v)

        def chunk(c, _):
            pltpu.sync_copy(ones_v, deg_sh.at[dst_v.at[c]], add=True)
            return 0
        lax.fori_loop(0, ec, chunk, 0)

        plsc.subcore_barrier()
        pltpu.sync_copy(deg_sh.at[pl.ds(base, RPT)],
                        deg_hbm.at[cid, pl.ds(base, RPT)])

    return deg_kernel


@functools.lru_cache(maxsize=None)
def _sc_layer(ec):
    @functools.partial(
        pl.kernel,
        out_type=jax.ShapeDtypeStruct((NC, NP, D), jnp.float32),
        mesh=plsc.VectorSubcoreMesh(**_MESH),
        scratch_types=[
            pltpu.VMEM((ec, CB), jnp.int32),
            pltpu.VMEM((ec, CB), jnp.int32),
            pltpu.VMEM((CB, D), jnp.float32),
            pltpu.VMEM_SHARED((NP, D), jnp.float32),
            pltpu.SemaphoreType.DMA,
        ],
    )
    def layer_kernel(y_hbm, src_hbm, dst_hbm, z_hbm,
                     src_v, dst_v, rows_v, z_sh, sem):
        cid = lax.axis_index("c")
        sid = lax.axis_index("s")
        w = cid * NS + sid
        base = sid * RPT

        def zrow(i, _):
            for j in range(D // LANES):
                rows_v[i, pl.ds(j * LANES, LANES)] = jnp.zeros((LANES,), jnp.float32)
            return 0
        lax.fori_loop(0, CB, zrow, 0)
        for k in range(RPT // CB):
            pltpu.sync_copy(rows_v, z_sh.at[pl.ds(base + k * CB, CB)])
        plsc.subcore_barrier()

        pltpu.sync_copy(src_hbm.at[w], src_v)
        pltpu.sync_copy(dst_hbm.at[w], dst_v)

        def chunk(c, _):
            pltpu.async_copy(y_hbm.at[src_v.at[c]], rows_v, sem).wait()
            pltpu.sync_copy(rows_v, z_sh.at[dst_v.at[c]], add=True)
            return 0
        lax.fori_loop(0, ec, chunk, 0)

        plsc.subcore_barrier()
        pltpu.sync_copy(z_sh.at[pl.ds(base, RPT)],
                        z_hbm.at[cid, pl.ds(base, RPT)])

    return layer_kernel


def _tc_first(x, W1, deg0, deg1):
    def body(x_ref, w_ref, d0_ref, d1_ref, y_ref, dis_ref):
        d = d0_ref[:, 0:1] + d1_ref[:, 0:1] + 1.0
        dis = lax.rsqrt(d)
        h = jnp.dot(x_ref[...], w_ref[...], preferred_element_type=jnp.float32)
        y_ref[...] = h * dis
        dis_ref[...] = jnp.broadcast_to(dis, (BR, D))

    return pl.pallas_call(
        body,
        grid=(G,),
        in_specs=[
            pl.BlockSpec((BR, D), lambda i: (i, 0)),
            pl.BlockSpec((D, D), lambda i: (0, 0)),
            pl.BlockSpec((BR, LANES), lambda i: (i, 0)),
            pl.BlockSpec((BR, LANES), lambda i: (i, 0)),
        ],
        out_specs=[
            pl.BlockSpec((BR, D), lambda i: (i, 0)),
            pl.BlockSpec((BR, D), lambda i: (i, 0)),
        ],
        out_shape=[
            jax.ShapeDtypeStruct((NP, D), jnp.float32),
            jax.ShapeDtypeStruct((NP, D), jnp.float32),
        ],
    )(x, W1, deg0, deg1)


def _tc_mid(z0, z1, y, dis, b, W):
    def body(z0_ref, z1_ref, y_ref, dis_ref, b_ref, w_ref, o_ref):
        i = pl.program_id(0)
        t = dis_ref[...] * (z0_ref[...] + z1_ref[...] + y_ref[...]) + b_ref[...]
        t = jnp.maximum(t, 0.0)
        rows = i * BR + lax.broadcasted_iota(jnp.int32, (BR, 1), 0)
        t = jnp.where(rows < N, t, 0.0)
        o_ref[...] = dis_ref[...] * jnp.dot(
            t, w_ref[...], preferred_element_type=jnp.float32)

    return pl.pallas_call(
        body,
        grid=(G,),
        in_specs=[
            pl.BlockSpec((BR, D), lambda i: (i, 0)),
            pl.BlockSpec((BR, D), lambda i: (i, 0)),
            pl.BlockSpec((BR, D), lambda i: (i, 0)),
            pl.BlockSpec((BR, D), lambda i: (i, 0)),
            pl.BlockSpec((1, D), lambda i: (0, 0)),
            pl.BlockSpec((D, D), lambda i: (0, 0)),
        ],
        out_specs=pl.BlockSpec((BR, D), lambda i: (i, 0)),
        out_shape=jax.ShapeDtypeStruct((NP, D), jnp.float32),
    )(z0, z1, y, dis, b, W)


def _tc_last(z0, z1, y, dis, b, Wp, bp):
    def body(z0_ref, z1_ref, y_ref, dis_ref, b_ref, wp_ref, bp_ref,
             o_ref, acc_ref):
        i = pl.program_id(0)

        @pl.when(i == 0)
        def _():
            acc_ref[...] = jnp.zeros_like(acc_ref)

        t = dis_ref[...] * (z0_ref[...] + z1_ref[...] + y_ref[...]) + b_ref[...]
        t = jnp.maximum(t, 0.0)
        rows = i * BR + lax.broadcasted_iota(jnp.int32, (BR, 1), 0)
        t = jnp.where(rows < N, t, 0.0)
        acc_ref[...] += jnp.sum(t, axis=0, keepdims=True)

        @pl.when(i == G - 1)
        def _():
            o_ref[...] = jnp.dot(acc_ref[...] * (1.0 / N), wp_ref[...],
                                 preferred_element_type=jnp.float32) + bp_ref[...]

    return pl.pallas_call(
        body,
        grid=(G,),
        in_specs=[
            pl.BlockSpec((BR, D), lambda i: (i, 0)),
            pl.BlockSpec((BR, D), lambda i: (i, 0)),
            pl.BlockSpec((BR, D), lambda i: (i, 0)),
            pl.BlockSpec((BR, D), lambda i: (i, 0)),
            pl.BlockSpec((1, D), lambda i: (0, 0)),
            pl.BlockSpec((D, D), lambda i: (0, 0)),
            pl.BlockSpec((1, D), lambda i: (0, 0)),
        ],
        out_specs=pl.BlockSpec((1, D), lambda i: (0, 0)),
        out_shape=jax.ShapeDtypeStruct((1, D), jnp.float32),
        scratch_shapes=[pltpu.VMEM((1, D), jnp.float32)],
    )(z0, z1, y, dis, b, Wp, bp)


def kernel(x, edge_index, W1, b1, W2, b2, Wp, bp):
    e = edge_index.shape[1]
    ec = -(-e // (NW * CB))
    ep = NW * CB * ec
    pad = ep - e
    pad_idx = (N + (jnp.arange(pad, dtype=jnp.int32) % (NP - N)))
    src_p = jnp.concatenate([edge_index[0], pad_idx]).reshape(NW, ec, CB)
    dst_p = jnp.concatenate([edge_index[1], pad_idx]).reshape(NW, ec, CB)
    x_p = jnp.concatenate([x, jnp.zeros((NP - N, D), x.dtype)], axis=0)

    degp = _sc_deg(ec)(dst_p)
    y1, dis = _tc_first(x_p, W1, degp[0], degp[1])
    z1 = _sc_layer(ec)(y1, src_p, dst_p)
    y2 = _tc_mid(z1[0], z1[1], y1, dis, b1.reshape(1, D), W2)
    z2 = _sc_layer(ec)(y2, src_p, dst_p)
    return _tc_last(z2[0], z2[1], y2, dis, b2.reshape(1, D), Wp,
                    bp.reshape(1, D))


# speedup vs baseline: 27.6839x; 1.2701x over previous
import functools

import jax
import jax.numpy as jnp
from jax import lax
from jax.experimental import pallas as pl
from jax.experimental.pallas import tpu as pltpu
from jax.experimental.pallas import tpu_sc as plsc

N = 10000
D = 128
NP = 10112
NC, NS, LANES = 2, 16, 16
NW = NC * NS
CB = 128
RPT = NP // NS
BR = 2528
G = NP // BR

_MESH = dict(core_axis_name="c", subcore_axis_name="s")


def _zero_slab(buf, sh, base):
    full, rem = divmod(RPT, CB)
    for k in range(full):
        pltpu.sync_copy(buf, sh.at[pl.ds(base + k * CB, CB)])
    if rem:
        pltpu.sync_copy(buf.at[pl.ds(0, rem)],
                        sh.at[pl.ds(base + full * CB, rem)])


@functools.lru_cache(maxsize=None)
def _sc_deg(ec):
    @functools.partial(
        pl.kernel,
        out_type=jax.ShapeDtypeStruct((NC, NP, LANES), jnp.float32),
        mesh=plsc.VectorSubcoreMesh(**_MESH),
        scratch_types=[
            pltpu.VMEM((ec, CB), jnp.int32),
            pltpu.VMEM((CB, LANES), jnp.float32),
            pltpu.VMEM_SHARED((NP, LANES), jnp.float32),
        ],
    )
    def deg_kernel(dst_hbm, deg_hbm, dst_v, ones_v, deg_sh):
        cid = lax.axis_index("c")
        sid = lax.axis_index("s")
        w = cid * NS + sid
        base = sid * RPT

        def fill(val):
            def body(i, _):
                ones_v[i] = jnp.full((LANES,), val, jnp.float32)
                return 0
            lax.fori_loop(0, CB, body, 0)

        fill(0.0)
        _zero_slab(ones_v, deg_sh, base)
        fill(1.0)
        plsc.subcore_barrier()

        pltpu.sync_copy(dst_hbm.at[w], dst_v)

        def chunk(c, _):
            pltpu.sync_copy(ones_v, deg_sh.at[dst_v.at[c]], add=True)
            return 0
        lax.fori_loop(0, ec, chunk, 0)

        plsc.subcore_barrier()
        pltpu.sync_copy(deg_sh.at[pl.ds(base, RPT)],
                        deg_hbm.at[cid, pl.ds(base, RPT)])

    return deg_kernel


@functools.lru_cache(maxsize=None)
def _sc_layer(ec):
    assert ec % 2 == 0 and ec >= 6

    @functools.partial(
        pl.kernel,
        out_type=jax.ShapeDtypeStruct((NC, NP, D), jnp.float32),
        mesh=plsc.VectorSubcoreMesh(**_MESH),
        scratch_types=[
            pltpu.VMEM((2, CB), jnp.int32),
            pltpu.VMEM((2, CB), jnp.int32),
            pltpu.VMEM((2, CB, D), jnp.float32),
            pltpu.VMEM_SHARED((NP, D), jnp.float32),
        ] + [pltpu.SemaphoreType.DMA] * 6,
    )
    def layer_kernel(y_hbm, src_hbm, dst_hbm, z_hbm,
                     src_v, dst_v, rows_v, z_sh, *sems):
        rsem = sems[0:2]
        ssem = sems[2:4]
        dsem = sems[4:6]
        cid = lax.axis_index("c")
        sid = lax.axis_index("s")
        w = cid * NS + sid
        base = sid * RPT

        def zrow(i, _):
            for j in range(D // LANES):
                rows_v[0, i, pl.ds(j * LANES, LANES)] = jnp.zeros(
                    (LANES,), jnp.float32)
            return 0
        lax.fori_loop(0, CB, zrow, 0)
        _zero_slab(rows_v.at[0], z_sh, base)
        plsc.subcore_barrier()

        def src_cp(b, c):
            return pltpu.make_async_copy(src_hbm.at[w, c], src_v.at[b],
                                         ssem[b])

        def dst_cp(b, c):
            return pltpu.make_async_copy(dst_hbm.at[w, c], dst_v.at[b],
                                         dsem[b])

        def rows_cp(b, c):
            del c
            return pltpu.make_async_copy(y_hbm.at[src_v.at[b]],
                                         rows_v.at[b], rsem[b])

        def do_chunk(c, b, tail=0):
            if tail < 2:
                src_cp(b ^ 1, c + 1).wait()
            rows_cp(b, c).wait()
            if tail < 2:
                rows_cp(b ^ 1, c + 1).start()
            if not tail:
                src_cp(b, c + 2).start()
            dst_cp(b, c).wait()
            pltpu.sync_copy(rows_v.at[b], z_sh.at[dst_v.at[b]], add=True)
            if not tail:
                dst_cp(b, c + 2).start()

        src_cp(0, 0).start()
        dst_cp(0, 0).start()
        src_cp(1, 1).start()
        dst_cp(1, 1).start()
        src_cp(0, 0).wait()
        rows_cp(0, 0).start()

        def group(g, _):
            c0 = g * 2
            do_chunk(c0, 0)
            do_chunk(c0 + 1, 1)
            return 0
        lax.fori_loop(0, ec // 2 - 1, group, 0)

        do_chunk(ec - 2, 0, tail=1)
        do_chunk(ec - 1, 1, tail=2)

        plsc.subcore_barrier()
        pltpu.sync_copy(z_sh.at[pl.ds(base, RPT)],
                        z_hbm.at[cid, pl.ds(base, RPT)])

    return layer_kernel


def _tc_first(x, W1, deg0, deg1):
    def body(x_ref, w_ref, d0_ref, d1_ref, y_ref, dis_ref):
        d = d0_ref[:, 0:1] + d1_ref[:, 0:1] + 1.0
        dis = lax.rsqrt(d)
        h = jnp.dot(x_ref[...], w_ref[...], preferred_element_type=jnp.float32)
        y_ref[...] = h * dis
        dis_ref[...] = jnp.broadcast_to(dis, (BR, D))

    return pl.pallas_call(
        body,
        grid=(G,),
        in_specs=[
            pl.BlockSpec((BR, D), lambda i: (i, 0)),
            pl.BlockSpec((D, D), lambda i: (0, 0)),
            pl.BlockSpec((BR, LANES), lambda i: (i, 0)),
            pl.BlockSpec((BR, LANES), lambda i: (i, 0)),
        ],
        out_specs=[
            pl.BlockSpec((BR, D), lambda i: (i, 0)),
            pl.BlockSpec((BR, D), lambda i: (i, 0)),
        ],
        out_shape=[
            jax.ShapeDtypeStruct((NP, D), jnp.float32),
            jax.ShapeDtypeStruct((NP, D), jnp.float32),
        ],
    )(x, W1, deg0, deg1)


def _tc_mid(z0, z1, y, dis, b, W):
    def body(z0_ref, z1_ref, y_ref, dis_ref, b_ref, w_ref, o_ref):
        i = pl.program_id(0)
        t = dis_ref[...] * (z0_ref[...] + z1_ref[...] + y_ref[...]) + b_ref[...]
        t = jnp.maximum(t, 0.0)
        rows = i * BR + lax.broadcasted_iota(jnp.int32, (BR, 1), 0)
        t = jnp.where(rows < N, t, 0.0)
        o_ref[...] = dis_ref[...] * jnp.dot(
            t, w_ref[...], preferred_element_type=jnp.float32)

    return pl.pallas_call(
        body,
        grid=(G,),
        in_specs=[
            pl.BlockSpec((BR, D), lambda i: (i, 0)),
            pl.BlockSpec((BR, D), lambda i: (i, 0)),
            pl.BlockSpec((BR, D), lambda i: (i, 0)),
            pl.BlockSpec((BR, D), lambda i: (i, 0)),
            pl.BlockSpec((1, D), lambda i: (0, 0)),
            pl.BlockSpec((D, D), lambda i: (0, 0)),
        ],
        out_specs=pl.BlockSpec((BR, D), lambda i: (i, 0)),
        out_shape=jax.ShapeDtypeStruct((NP, D), jnp.float32),
    )(z0, z1, y, dis, b, W)


def _tc_last(z0, z1, y, dis, b, Wp, bp):
    def body(z0_ref, z1_ref, y_ref, dis_ref, b_ref, wp_ref, bp_ref,
             o_ref, acc_ref):
        i = pl.program_id(0)

        @pl.when(i == 0)
        def _():
            acc_ref[...] = jnp.zeros_like(acc_ref)

        t = dis_ref[...] * (z0_ref[...] + z1_ref[...] + y_ref[...]) + b_ref[...]
        t = jnp.maximum(t, 0.0)
        rows = i * BR + lax.broadcasted_iota(jnp.int32, (BR, 1), 0)
        t = jnp.where(rows < N, t, 0.0)
        acc_ref[...] += jnp.sum(t, axis=0, keepdims=True)

        @pl.when(i == G - 1)
        def _():
            o_ref[...] = jnp.dot(acc_ref[...] * (1.0 / N), wp_ref[...],
                                 preferred_element_type=jnp.float32) + bp_ref[...]

    return pl.pallas_call(
        body,
        grid=(G,),
        in_specs=[
            pl.BlockSpec((BR, D), lambda i: (i, 0)),
            pl.BlockSpec((BR, D), lambda i: (i, 0)),
            pl.BlockSpec((BR, D), lambda i: (i, 0)),
            pl.BlockSpec((BR, D), lambda i: (i, 0)),
            pl.BlockSpec((1, D), lambda i: (0, 0)),
            pl.BlockSpec((D, D), lambda i: (0, 0)),
            pl.BlockSpec((1, D), lambda i: (0, 0)),
        ],
        out_specs=pl.BlockSpec((1, D), lambda i: (0, 0)),
        out_shape=jax.ShapeDtypeStruct((1, D), jnp.float32),
        scratch_shapes=[pltpu.VMEM((1, D), jnp.float32)],
    )(z0, z1, y, dis, b, Wp, bp)


def kernel(x, edge_index, W1, b1, W2, b2, Wp, bp):
    e = edge_index.shape[1]
    ec = -(-e // (NW * CB))
    ec = -(-ec // 2) * 2
    ep = NW * CB * ec
    pad_idx = N + (jnp.arange(ep - e, dtype=jnp.int32) % (NP - N))
    src_p = jnp.concatenate([edge_index[0], pad_idx]).reshape(NW, ec, CB)
    dst_p = jnp.concatenate([edge_index[1], pad_idx]).reshape(NW, ec, CB)
    x_p = jnp.concatenate([x, jnp.zeros((NP - N, D), x.dtype)], axis=0)

    degp = _sc_deg(ec)(dst_p)
    y1, dis = _tc_first(x_p, W1, degp[0], degp[1])
    z1 = _sc_layer(ec)(y1, src_p, dst_p)
    y2 = _tc_mid(z1[0], z1[1], y1, dis, b1.reshape(1, D), W2)
    z2 = _sc_layer(ec)(y2, src_p, dst_p)
    return _tc_last(z2[0], z2[1], y2, dis, b2.reshape(1, D), Wp,
                    bp.reshape(1, D))


# speedup vs baseline: 27.9107x; 1.0082x over previous
import functools

import jax
import jax.numpy as jnp
from jax import lax
from jax.experimental import pallas as pl
from jax.experimental.pallas import tpu as pltpu
from jax.experimental.pallas import tpu_sc as plsc

N = 10000
D = 128
NP = 10112
NC, NS, LANES = 2, 16, 16
NW = NC * NS
CB = 128
RPT = NP // NS
BR = 2528
G = NP // BR

_MESH = dict(core_axis_name="c", subcore_axis_name="s")


def _zero_slab(buf, sh, base):
    full, rem = divmod(RPT, CB)
    for k in range(full):
        pltpu.sync_copy(buf, sh.at[pl.ds(base + k * CB, CB)])
    if rem:
        pltpu.sync_copy(buf.at[pl.ds(0, rem)],
                        sh.at[pl.ds(base + full * CB, rem)])


@functools.lru_cache(maxsize=None)
def _sc_deg(ec):
    @functools.partial(
        pl.kernel,
        out_type=jax.ShapeDtypeStruct((NC, NP, LANES), jnp.float32),
        mesh=plsc.VectorSubcoreMesh(**_MESH),
        scratch_types=[
            pltpu.VMEM((ec, CB), jnp.int32),
            pltpu.VMEM((CB, LANES), jnp.float32),
            pltpu.VMEM_SHARED((NP, LANES), jnp.float32),
            pltpu.SemaphoreType.DMA,
        ],
    )
    def deg_kernel(dst_hbm, deg_hbm, dst_v, ones_v, deg_sh, dgsem):
        cid = lax.axis_index("c")
        sid = lax.axis_index("s")
        w = cid * NS + sid
        base = sid * RPT

        def fill(val):
            def body(i, _):
                ones_v[i] = jnp.full((LANES,), val, jnp.float32)
                return 0
            lax.fori_loop(0, CB, body, 0)

        fill(0.0)
        _zero_slab(ones_v, deg_sh, base)
        fill(1.0)
        plsc.subcore_barrier()

        pltpu.sync_copy(dst_hbm.at[w], dst_v)

        kf = 16
        def batch(s, _):
            def fire(c, _):
                pltpu.async_copy(ones_v, deg_sh.at[dst_v.at[c]], dgsem,
                                 add=True)
                return 0
            lax.fori_loop(s, s + kf, fire, 0)
            def drain(c, _):
                pltpu.make_async_copy(ones_v, deg_sh.at[dst_v.at[c]],
                                      dgsem).wait()
                return 0
            lax.fori_loop(s, s + kf, drain, 0)
            return 0
        lax.fori_loop(0, ec // kf, lambda g, x: batch(g * kf, x), 0)
        def tail(c, _):
            pltpu.sync_copy(ones_v, deg_sh.at[dst_v.at[c]], add=True)
            return 0
        lax.fori_loop(ec // kf * kf, ec, tail, 0)

        plsc.subcore_barrier()
        pltpu.sync_copy(deg_sh.at[pl.ds(base, RPT)],
                        deg_hbm.at[cid, pl.ds(base, RPT)])

    return deg_kernel


@functools.lru_cache(maxsize=None)
def _sc_layer(ec):
    assert ec % 4 == 0 and ec >= 12

    @functools.partial(
        pl.kernel,
        out_type=jax.ShapeDtypeStruct((NC, NP, D), jnp.float32),
        mesh=plsc.VectorSubcoreMesh(**_MESH),
        scratch_types=[
            pltpu.VMEM((2, CB), jnp.int32),
            pltpu.VMEM((4, CB), jnp.int32),
            pltpu.VMEM((2, CB, D), jnp.float32),
            pltpu.VMEM_SHARED((NP, D), jnp.float32),
        ] + [pltpu.SemaphoreType.DMA] * 10,
    )
    def layer_kernel(y_hbm, src_hbm, dst_hbm, z_hbm,
                     src_v, dst_v, rows_v, z_sh, *sems):
        rsem = sems[0:2]
        ssem = sems[2:4]
        dsem = sems[4:8]
        csem = sems[8:10]
        cid = lax.axis_index("c")
        sid = lax.axis_index("s")
        w = cid * NS + sid
        base = sid * RPT

        def zrow(i, _):
            for j in range(D // LANES):
                rows_v[0, i, pl.ds(j * LANES, LANES)] = jnp.zeros(
                    (LANES,), jnp.float32)
            return 0
        lax.fori_loop(0, CB, zrow, 0)
        _zero_slab(rows_v.at[0], z_sh, base)
        plsc.subcore_barrier()

        def src_cp(b, c):
            return pltpu.make_async_copy(src_hbm.at[w, c], src_v.at[b],
                                         ssem[b])

        def dst_cp(q, c):
            return pltpu.make_async_copy(dst_hbm.at[w, c], dst_v.at[q],
                                         dsem[q])

        def rows_cp(b):
            return pltpu.make_async_copy(y_hbm.at[src_v.at[b]],
                                         rows_v.at[b], rsem[b])

        def scat_cp(b, q):
            return pltpu.make_async_copy(rows_v.at[b],
                                         z_sh.at[dst_v.at[q]], csem[b])

        def do_chunk(c, b, q, first=False, tail=0):
            rows_cp(b).wait()
            dst_cp(q, c).wait()
            pltpu.async_copy(rows_v.at[b], z_sh.at[dst_v.at[q]], csem[b],
                             add=True)
            if tail < 2:
                src_cp(b ^ 1, c + 1).wait()
                if not first:
                    scat_cp(b ^ 1, (q + 3) % 4).wait()
                rows_cp(b ^ 1).start()
            if not tail:
                src_cp(b, c + 2).start()
                dst_cp((q + 2) % 4, c + 2).start()

        src_cp(0, 0).start()
        dst_cp(0, 0).start()
        src_cp(1, 1).start()
        dst_cp(1, 1).start()
        src_cp(0, 0).wait()
        rows_cp(0).start()

        do_chunk(0, 0, 0, first=True)
        do_chunk(1, 1, 1)
        do_chunk(2, 0, 2)
        do_chunk(3, 1, 3)

        def group(g, _):
            c0 = g * 4
            do_chunk(c0, 0, 0)
            do_chunk(c0 + 1, 1, 1)
            do_chunk(c0 + 2, 0, 2)
            do_chunk(c0 + 3, 1, 3)
            return 0
        lax.fori_loop(1, ec // 4 - 1, group, 0)

        c0 = ec - 4
        do_chunk(c0, 0, 0)
        do_chunk(c0 + 1, 1, 1)
        do_chunk(c0 + 2, 0, 2, tail=1)
        do_chunk(c0 + 3, 1, 3, tail=2)
        scat_cp(0, 2).wait()
        scat_cp(1, 3).wait()

        plsc.subcore_barrier()
        pltpu.sync_copy(z_sh.at[pl.ds(base, RPT)],
                        z_hbm.at[cid, pl.ds(base, RPT)])

    return layer_kernel


def _tc_first(x, W1, deg0, deg1):
    def body(x_ref, w_ref, d0_ref, d1_ref, y_ref, dis_ref):
        d = d0_ref[:, 0:1] + d1_ref[:, 0:1] + 1.0
        dis = lax.rsqrt(d)
        h = jnp.dot(x_ref[...], w_ref[...], preferred_element_type=jnp.float32)
        y_ref[...] = h * dis
        dis_ref[...] = jnp.broadcast_to(dis, (BR, D))

    return pl.pallas_call(
        body,
        grid=(G,),
        in_specs=[
            pl.BlockSpec((BR, D), lambda i: (i, 0)),
            pl.BlockSpec((D, D), lambda i: (0, 0)),
            pl.BlockSpec((BR, LANES), lambda i: (i, 0)),
            pl.BlockSpec((BR, LANES), lambda i: (i, 0)),
        ],
        out_specs=[
            pl.BlockSpec((BR, D), lambda i: (i, 0)),
            pl.BlockSpec((BR, D), lambda i: (i, 0)),
        ],
        out_shape=[
            jax.ShapeDtypeStruct((NP, D), jnp.float32),
            jax.ShapeDtypeStruct((NP, D), jnp.float32),
        ],
    )(x, W1, deg0, deg1)


def _tc_mid(z0, z1, y, dis, b, W):
    def body(z0_ref, z1_ref, y_ref, dis_ref, b_ref, w_ref, o_ref):
        i = pl.program_id(0)
        t = dis_ref[...] * (z0_ref[...] + z1_ref[...] + y_ref[...]) + b_ref[...]
        t = jnp.maximum(t, 0.0)
        rows = i * BR + lax.broadcasted_iota(jnp.int32, (BR, 1), 0)
        t = jnp.where(rows < N, t, 0.0)
        o_ref[...] = dis_ref[...] * jnp.dot(
            t, w_ref[...], preferred_element_type=jnp.float32)

    return pl.pallas_call(
        body,
        grid=(G,),
        in_specs=[
            pl.BlockSpec((BR, D), lambda i: (i, 0)),
            pl.BlockSpec((BR, D), lambda i: (i, 0)),
            pl.BlockSpec((BR, D), lambda i: (i, 0)),
            pl.BlockSpec((BR, D), lambda i: (i, 0)),
            pl.BlockSpec((1, D), lambda i: (0, 0)),
            pl.BlockSpec((D, D), lambda i: (0, 0)),
        ],
        out_specs=pl.BlockSpec((BR, D), lambda i: (i, 0)),
        out_shape=jax.ShapeDtypeStruct((NP, D), jnp.float32),
    )(z0, z1, y, dis, b, W)


def _tc_last(z0, z1, y, dis, b, Wp, bp):
    def body(z0_ref, z1_ref, y_ref, dis_ref, b_ref, wp_ref, bp_ref,
             o_ref, acc_ref):
        i = pl.program_id(0)

        @pl.when(i == 0)
        def _():
            acc_ref[...] = jnp.zeros_like(acc_ref)

        t = dis_ref[...] * (z0_ref[...] + z1_ref[...] + y_ref[...]) + b_ref[...]
        t = jnp.maximum(t, 0.0)
        rows = i * BR + lax.broadcasted_iota(jnp.int32, (BR, 1), 0)
        t = jnp.where(rows < N, t, 0.0)
        acc_ref[...] += jnp.sum(t, axis=0, keepdims=True)

        @pl.when(i == G - 1)
        def _():
            o_ref[...] = jnp.dot(acc_ref[...] * (1.0 / N), wp_ref[...],
                                 preferred_element_type=jnp.float32) + bp_ref[...]

    return pl.pallas_call(
        body,
        grid=(G,),
        in_specs=[
            pl.BlockSpec((BR, D), lambda i: (i, 0)),
            pl.BlockSpec((BR, D), lambda i: (i, 0)),
            pl.BlockSpec((BR, D), lambda i: (i, 0)),
            pl.BlockSpec((BR, D), lambda i: (i, 0)),
            pl.BlockSpec((1, D), lambda i: (0, 0)),
            pl.BlockSpec((D, D), lambda i: (0, 0)),
            pl.BlockSpec((1, D), lambda i: (0, 0)),
        ],
        out_specs=pl.BlockSpec((1, D), lambda i: (0, 0)),
        out_shape=jax.ShapeDtypeStruct((1, D), jnp.float32),
        scratch_shapes=[pltpu.VMEM((1, D), jnp.float32)],
    )(z0, z1, y, dis, b, Wp, bp)


def kernel(x, edge_index, W1, b1, W2, b2, Wp, bp):
    e = edge_index.shape[1]
    ec = -(-e // (NW * CB))
    ec = -(-ec // 2) * 2
    ep = NW * CB * ec
    pad_idx = N + (jnp.arange(ep - e, dtype=jnp.int32) % (NP - N))
    src_p = jnp.concatenate([edge_index[0], pad_idx]).reshape(NW, ec, CB)
    dst_p = jnp.concatenate([edge_index[1], pad_idx]).reshape(NW, ec, CB)
    x_p = jnp.concatenate([x, jnp.zeros((NP - N, D), x.dtype)], axis=0)

    degp = _sc_deg(ec)(dst_p)
    y1, dis = _tc_first(x_p, W1, degp[0], degp[1])
    z1 = _sc_layer(ec)(y1, src_p, dst_p)
    y2 = _tc_mid(z1[0], z1[1], y1, dis, b1.reshape(1, D), W2)
    z2 = _sc_layer(ec)(y2, src_p, dst_p)
    return _tc_last(z2[0], z2[1], y2, dis, b2.reshape(1, D), Wp,
                    bp.reshape(1, D))
